# baseline (speedup 1.0000x reference)
import jax
import jax.numpy as jnp
from jax import lax
from jax.experimental import pallas as pl
from jax.experimental.pallas import tpu as pltpu
from jax.experimental.pallas import tpu_sc as plsc

B = 16384
HID = 1024
HOPS = 8
NEXP = 4

_NC = 2
_NS = 16
_NW = _NC * _NS
_RPW = B // _NW
_CH = 64
_NCHUNK = _RPW // _CH

_R = 512
_NSTEP = B // _R


def _sc_gather_body(ent_hbm, reli_hbm, subs_hbm, rels_hbm, out_sub, out_reli,
                    idx_v, erows_v, rrows_v, sem):
    wid = lax.axis_index("s") * _NC + lax.axis_index("c")
    base = wid * _RPW
    for tab, ind, out, buf in ((ent_hbm, subs_hbm, out_sub, erows_v),
                               (reli_hbm, rels_hbm, out_reli, rrows_v)):
        for c in range(_NCHUNK):
            off = base + c * _CH
            pltpu.sync_copy(ind.at[pl.ds(off, _CH)], idx_v)
            pltpu.async_copy(tab.at[idx_v], buf, sem).wait()
            pltpu.sync_copy(buf, out.at[pl.ds(off, _CH)])


def _gather_rows(entity_embed, reli, subs, rels):
    mesh = plsc.VectorSubcoreMesh(core_axis_name="c", subcore_axis_name="s",
                                  num_cores=_NC, num_subcores=_NS)
    return pl.kernel(
        _sc_gather_body,
        out_type=(jax.ShapeDtypeStruct((B, HID), jnp.float32),
                  jax.ShapeDtypeStruct((B, HID // 2), jnp.int32)),
        mesh=mesh,
        scratch_types=(pltpu.VMEM((_CH,), jnp.int32),
                       pltpu.VMEM((_CH, HID), jnp.float32),
                       pltpu.VMEM((_CH, HID // 2), jnp.int32),
                       pltpu.SemaphoreType.DMA),
    )(entity_embed, reli, subs, rels)


def _tc_body(sub_ref, rel_ref, W1_ref, b1_ref, W2_ref, b2_ref, hop_ref,
             wn_ref, noise_ref, G_ref, Q_ref, acc_ref):
    i = pl.program_id(0)

    @pl.when(i == 0)
    def _():
        acc_ref[...] = jnp.zeros_like(acc_ref)

    dn = (((1,), (0,)), ((), ()))
    z = lax.dot_general(sub_ref[...].astype(jnp.bfloat16), W1_ref[0:HID, :],
                        dn, preferred_element_type=jnp.float32)
    z = z + lax.dot_general(rel_ref[...], W1_ref[HID:2 * HID, :], dn,
                            preferred_element_type=jnp.float32)
    z = z + b1_ref[...]
    h = jnp.maximum(z, 0.0)
    acc_ref[...] += jnp.sum(h, axis=0, keepdims=True)

    @pl.when(i == _NSTEP - 1)
    def _():
        c_i = acc_ref[...] * (1.0 / B)
        c_i = lax.dot_general(c_i, W2_ref[...], dn,
                              preferred_element_type=jnp.float32) + b2_ref[...]
        q = lax.dot_general(c_i, hop_ref[...], (((1,), (1,)), ((), ())),
                            preferred_element_type=jnp.float32)
        sx = jnp.sum(c_i * wn_ref[...])
        sigma = jnp.maximum(sx, 0.0) + jnp.log1p(jnp.exp(-jnp.abs(sx)))
        q = q + noise_ref[...] * sigma

        iot = lax.broadcasted_iota(jnp.int32, (1, HOPS), 1)
        rank = jnp.zeros((1, HOPS), jnp.int32)
        for j in range(HOPS):
            qj = q[0, j]
            beats = (qj > q) | ((qj == q) & (j < iot))
            rank = rank + beats.astype(jnp.int32)
        sel = rank < NEXP
        m = jnp.max(jnp.where(sel, q, -1e30))
        e = jnp.where(sel, jnp.exp(q - m), 0.0)
        G_ref[...] = e / jnp.sum(e)
        Q_ref[...] = q


def _moe_head(sub_rows, rel_rows, W1, b1, W2, b2, hop_embed, wn_row, noise_row):
    g, q = pl.pallas_call(
        _tc_body,
        grid=(_NSTEP,),
        in_specs=[
            pl.BlockSpec((_R, HID), lambda i: (i, 0)),
            pl.BlockSpec((_R, HID), lambda i: (i, 0)),
            pl.BlockSpec((2 * HID, HID), lambda i: (0, 0)),
            pl.BlockSpec((1, HID), lambda i: (0, 0)),
            pl.BlockSpec((HID, HID), lambda i: (0, 0)),
            pl.BlockSpec((1, HID), lambda i: (0, 0)),
            pl.BlockSpec((HOPS, HID), lambda i: (0, 0)),
            pl.BlockSpec((1, HID), lambda i: (0, 0)),
            pl.BlockSpec((1, HOPS), lambda i: (0, 0)),
        ],
        out_specs=[
            pl.BlockSpec((1, HOPS), lambda i: (0, 0)),
            pl.BlockSpec((1, HOPS), lambda i: (0, 0)),
        ],
        out_shape=[
            jax.ShapeDtypeStruct((1, HOPS), jnp.float32),
            jax.ShapeDtypeStruct((1, HOPS), jnp.float32),
        ],
        scratch_shapes=[pltpu.VMEM((1, HID), jnp.float32)],
        compiler_params=pltpu.CompilerParams(
            dimension_semantics=("arbitrary",)),
    )(sub_rows, rel_rows, W1, b1, W2, b2, hop_embed, wn_row, noise_row)
    return g, q


def kernel(subs, rels, entity_embed, relation_embed, hop_embed, W1, b1, W2,
           b2, w_n, noise_eps):
    nrel = relation_embed.shape[0]
    reli = lax.bitcast_convert_type(
        relation_embed.astype(jnp.bfloat16).reshape(nrel, HID // 2, 2),
        jnp.int32)
    sub_rows, reli_rows = _gather_rows(entity_embed, reli, subs, rels)
    rel_rows = lax.bitcast_convert_type(reli_rows,
                                        jnp.bfloat16).reshape(B, HID)
    g, q = _moe_head(sub_rows, rel_rows,
                     W1.astype(jnp.bfloat16),
                     b1.reshape(1, HID), W2, b2.reshape(1, HID),
                     hop_embed, w_n.reshape(1, HID),
                     noise_eps.reshape(1, HOPS))
    return (g.reshape(HOPS), q.reshape(HOPS))


# speedup vs baseline: 2.3088x; 2.3088x over previous
import jax
import jax.numpy as jnp
from jax import lax
from jax.experimental import pallas as pl
from jax.experimental.pallas import tpu as pltpu
from jax.experimental.pallas import tpu_sc as plsc

B = 16384
HID = 1024
HOPS = 8
NEXP = 4

_NCK = 4
_BC = B // _NCK

_NC = 2
_NS = 16
_NW = _NC * _NS
_RPW = _BC // _NW
_CH = 64
_NCHUNK = _RPW // _CH

_R = 512
_NSTEP = _BC // _R


def _sc_gather_body(ent_hbm, rel_hbm, subs_hbm, rels_hbm, out_sub, out_rel,
                    idx_v, rows_v, sem):
    wid = lax.axis_index("s") * _NC + lax.axis_index("c")
    base = wid * _RPW
    for tab, ind, out in ((ent_hbm, subs_hbm, out_sub),
                          (rel_hbm, rels_hbm, out_rel)):
        for c in range(_NCHUNK):
            off = base + c * _CH
            pltpu.sync_copy(ind.at[pl.ds(off, _CH)], idx_v)
            pltpu.async_copy(tab.at[idx_v], rows_v, sem).wait()
            pltpu.sync_copy(rows_v, out.at[pl.ds(off, _CH)])


def _gather_rows(entity_embed, relation_embed, subs_c, rels_c):
    mesh = plsc.VectorSubcoreMesh(core_axis_name="c", subcore_axis_name="s",
                                  num_cores=_NC, num_subcores=_NS)
    return pl.kernel(
        _sc_gather_body,
        out_type=(jax.ShapeDtypeStruct((_BC, HID), jnp.float32),
                  jax.ShapeDtypeStruct((_BC, HID), jnp.float32)),
        mesh=mesh,
        scratch_types=(pltpu.VMEM((_CH,), jnp.int32),
                       pltpu.VMEM((_CH, HID), jnp.float32),
                       pltpu.SemaphoreType.DMA),
    )(entity_embed, relation_embed, subs_c, rels_c)


def _tc_partial_body(sub_ref, rel_ref, W1_ref, b1_ref, psum_ref, acc_ref):
    i = pl.program_id(0)

    @pl.when(i == 0)
    def _():
        acc_ref[...] = jnp.zeros_like(acc_ref)

    dn = (((1,), (0,)), ((), ()))
    z = lax.dot_general(sub_ref[...].astype(jnp.bfloat16), W1_ref[0:HID, :],
                        dn, preferred_element_type=jnp.float32)
    z = z + lax.dot_general(rel_ref[...].astype(jnp.bfloat16),
                            W1_ref[HID:2 * HID, :], dn,
                            preferred_element_type=jnp.float32)
    z = z + b1_ref[...]
    h = jnp.maximum(z, 0.0)
    acc_ref[...] += jnp.sum(h, axis=0, keepdims=True)

    @pl.when(i == _NSTEP - 1)
    def _():
        psum_ref[...] = acc_ref[...]


def _tc_partial(sub_c, rel_c, W1, b1):
    return pl.pallas_call(
        _tc_partial_body,
        grid=(_NSTEP,),
        in_specs=[
            pl.BlockSpec((_R, HID), lambda i: (i, 0)),
            pl.BlockSpec((_R, HID), lambda i: (i, 0)),
            pl.BlockSpec((2 * HID, HID), lambda i: (0, 0)),
            pl.BlockSpec((1, HID), lambda i: (0, 0)),
        ],
        out_specs=pl.BlockSpec((1, HID), lambda i: (0, 0)),
        out_shape=jax.ShapeDtypeStruct((1, HID), jnp.float32),
        scratch_shapes=[pltpu.VMEM((1, HID), jnp.float32)],
        compiler_params=pltpu.CompilerParams(
            dimension_semantics=("arbitrary",)),
    )(sub_c, rel_c, W1, b1)


def _tc_final_body(p_ref, W2_ref, b2_ref, hop_ref, wn_ref, noise_ref,
                   G_ref, Q_ref):
    dn = (((1,), (0,)), ((), ()))
    c_i = jnp.sum(p_ref[...], axis=0, keepdims=True) * (1.0 / B)
    c_i = lax.dot_general(c_i, W2_ref[...], dn,
                          preferred_element_type=jnp.float32) + b2_ref[...]
    q = lax.dot_general(c_i, hop_ref[...], (((1,), (1,)), ((), ())),
                        preferred_element_type=jnp.float32)
    sx = jnp.sum(c_i * wn_ref[...])
    sigma = jnp.maximum(sx, 0.0) + jnp.log1p(jnp.exp(-jnp.abs(sx)))
    q = q + noise_ref[...] * sigma

    iot = lax.broadcasted_iota(jnp.int32, (1, HOPS), 1)
    rank = jnp.zeros((1, HOPS), jnp.int32)
    for j in range(HOPS):
        qj = q[0, j]
        beats = (qj > q) | ((qj == q) & (j < iot))
        rank = rank + beats.astype(jnp.int32)
    sel = rank < NEXP
    m = jnp.max(jnp.where(sel, q, -1e30))
    e = jnp.where(sel, jnp.exp(q - m), 0.0)
    G_ref[...] = e / jnp.sum(e)
    Q_ref[...] = q


def _tc_final(psums, W2, b2, hop_embed, wn_row, noise_row):
    return pl.pallas_call(
        _tc_final_body,
        in_specs=[
            pl.BlockSpec((_NCK, HID), lambda: (0, 0)),
            pl.BlockSpec((HID, HID), lambda: (0, 0)),
            pl.BlockSpec((1, HID), lambda: (0, 0)),
            pl.BlockSpec((HOPS, HID), lambda: (0, 0)),
            pl.BlockSpec((1, HID), lambda: (0, 0)),
            pl.BlockSpec((1, HOPS), lambda: (0, 0)),
        ],
        out_specs=[
            pl.BlockSpec((1, HOPS), lambda: (0, 0)),
            pl.BlockSpec((1, HOPS), lambda: (0, 0)),
        ],
        out_shape=[
            jax.ShapeDtypeStruct((1, HOPS), jnp.float32),
            jax.ShapeDtypeStruct((1, HOPS), jnp.float32),
        ],
    )(psums, W2, b2, hop_embed, wn_row, noise_row)


def kernel(subs, rels, entity_embed, relation_embed, hop_embed, W1, b1, W2,
           b2, w_n, noise_eps):
    W1b = W1.astype(jnp.bfloat16)
    b1r = b1.reshape(1, HID)
    psums = []
    for k in range(_NCK):
        subs_c = lax.slice(subs, (k * _BC,), ((k + 1) * _BC,))
        rels_c = lax.slice(rels, (k * _BC,), ((k + 1) * _BC,))
        sub_c, rel_c = _gather_rows(entity_embed, relation_embed,
                                    subs_c, rels_c)
        psums.append(_tc_partial(sub_c, rel_c, W1b, b1r))
    g, q = _tc_final(jnp.concatenate(psums, axis=0), W2,
                     b2.reshape(1, HID), hop_embed, w_n.reshape(1, HID),
                     noise_eps.reshape(1, HOPS))
    return (g.reshape(HOPS), q.reshape(HOPS))


# speedup vs baseline: 2.6192x; 1.1345x over previous
import jax
import jax.numpy as jnp
from jax import lax
from jax.experimental import pallas as pl
from jax.experimental.pallas import tpu as pltpu
from jax.experimental.pallas import tpu_sc as plsc

B = 16384
HID = 1024
HOPS = 8
NEXP = 4

_NCK = 4
_BC = B // _NCK

_NC = 2
_NS = 16
_NW = _NC * _NS
_RPW = _BC // _NW
_CH = 64
_NCHUNK = _RPW // _CH

_R = 512
_NSTEP = _BC // _R


def _sc_gather_body(ent_hbm, rel_hbm, subs_hbm, rels_hbm, out_sub, out_rel,
                    idx_v, rows_v, rrows_v, sem):
    wid = lax.axis_index("s") * _NC + lax.axis_index("c")
    base = wid * _RPW
    for tab, ind, out, buf in ((ent_hbm, subs_hbm, out_sub, rows_v),
                               (rel_hbm, rels_hbm, out_rel, rrows_v)):
        for c in range(_NCHUNK):
            off = base + c * _CH
            pltpu.sync_copy(ind.at[pl.ds(off, _CH)], idx_v)
            pltpu.async_copy(tab.at[idx_v], buf, sem).wait()
            pltpu.sync_copy(buf, out.at[pl.ds(off, _CH)])


def _gather_rows(entity_embed, relation_embed, subs_c, rels_c):
    mesh = plsc.VectorSubcoreMesh(core_axis_name="c", subcore_axis_name="s",
                                  num_cores=_NC, num_subcores=_NS)
    return pl.kernel(
        _sc_gather_body,
        out_type=(jax.ShapeDtypeStruct((_BC, HID), jnp.float32),
                  jax.ShapeDtypeStruct((_BC, HID // 2), jnp.int32)),
        mesh=mesh,
        scratch_types=(pltpu.VMEM((_CH,), jnp.int32),
                       pltpu.VMEM((_CH, HID), jnp.float32),
                       pltpu.VMEM((_CH, HID // 2), jnp.int32),
                       pltpu.SemaphoreType.DMA),
    )(entity_embed, relation_embed, subs_c, rels_c)


def _pack_body(x_ref, out_ref):
    a = x_ref[:, 0:HID // 2].astype(jnp.bfloat16).astype(jnp.float32)
    b = x_ref[:, HID // 2:HID].astype(jnp.bfloat16).astype(jnp.float32)
    ai = lax.bitcast_convert_type(a, jnp.int32)
    bi = lax.bitcast_convert_type(b, jnp.int32)
    out_ref[...] = (lax.shift_right_logical(ai, 16)
                    | (bi & jnp.int32(-65536)))


def _pack_rel(relation_embed):
    nrel = relation_embed.shape[0]
    nblk = (nrel + 127) // 128
    return pl.pallas_call(
        _pack_body,
        grid=(nblk,),
        in_specs=[pl.BlockSpec((128, HID), lambda i: (i, 0))],
        out_specs=pl.BlockSpec((128, HID // 2), lambda i: (i, 0)),
        out_shape=jax.ShapeDtypeStruct((nrel, HID // 2), jnp.int32),
    )(relation_embed)


def _tc_partial_body(sub_ref, rel_ref, W1_ref, b1_ref, psum_ref, acc_ref):
    i = pl.program_id(0)

    @pl.when(i == 0)
    def _():
        acc_ref[...] = jnp.zeros_like(acc_ref)

    dn = (((1,), (0,)), ((), ()))
    x = rel_ref[...]
    rel_a = lax.bitcast_convert_type(lax.shift_left(x, 16),
                                     jnp.float32).astype(jnp.bfloat16)
    rel_b = lax.bitcast_convert_type(x & jnp.int32(-65536),
                                     jnp.float32).astype(jnp.bfloat16)
    z = lax.dot_general(sub_ref[...].astype(jnp.bfloat16), W1_ref[0:HID, :],
                        dn, preferred_element_type=jnp.float32)
    z = z + lax.dot_general(rel_a, W1_ref[HID:HID + HID // 2, :], dn,
                            preferred_element_type=jnp.float32)
    z = z + lax.dot_general(rel_b, W1_ref[HID + HID // 2:2 * HID, :], dn,
                            preferred_element_type=jnp.float32)
    z = z + b1_ref[...]
    h = jnp.maximum(z, 0.0)
    acc_ref[...] += jnp.sum(h, axis=0, keepdims=True)

    @pl.when(i == _NSTEP - 1)
    def _():
        psum_ref[...] = acc_ref[...]


def _tc_partial(sub_c, rel_c, W1, b1):
    return pl.pallas_call(
        _tc_partial_body,
        grid=(_NSTEP,),
        in_specs=[
            pl.BlockSpec((_R, HID), lambda i: (i, 0)),
            pl.BlockSpec((_R, HID // 2), lambda i: (i, 0)),
            pl.BlockSpec((2 * HID, HID), lambda i: (0, 0)),
            pl.BlockSpec((1, HID), lambda i: (0, 0)),
        ],
        out_specs=pl.BlockSpec((1, HID), lambda i: (0, 0)),
        out_shape=jax.ShapeDtypeStruct((1, HID), jnp.float32),
        scratch_shapes=[pltpu.VMEM((1, HID), jnp.float32)],
        compiler_params=pltpu.CompilerParams(
            dimension_semantics=("arbitrary",)),
    )(sub_c, rel_c, W1, b1)


def _tc_final_body(p_ref, W2_ref, b2_ref, hop_ref, wn_ref, noise_ref,
                   G_ref, Q_ref):
    dn = (((1,), (0,)), ((), ()))
    c_i = jnp.sum(p_ref[...], axis=0, keepdims=True) * (1.0 / B)
    c_i = lax.dot_general(c_i, W2_ref[...], dn,
                          preferred_element_type=jnp.float32) + b2_ref[...]
    q = lax.dot_general(c_i, hop_ref[...], (((1,), (1,)), ((), ())),
                        preferred_element_type=jnp.float32)
    sx = jnp.sum(c_i * wn_ref[...])
    sigma = jnp.maximum(sx, 0.0) + jnp.log1p(jnp.exp(-jnp.abs(sx)))
    q = q + noise_ref[...] * sigma

    iot = lax.broadcasted_iota(jnp.int32, (1, HOPS), 1)
    rank = jnp.zeros((1, HOPS), jnp.int32)
    for j in range(HOPS):
        qj = q[0, j]
        beats = (qj > q) | ((qj == q) & (j < iot))
        rank = rank + beats.astype(jnp.int32)
    sel = rank < NEXP
    m = jnp.max(jnp.where(sel, q, -1e30))
    e = jnp.where(sel, jnp.exp(q - m), 0.0)
    G_ref[...] = e / jnp.sum(e)
    Q_ref[...] = q


def _tc_final(psums, W2, b2, hop_embed, wn_row, noise_row):
    return pl.pallas_call(
        _tc_final_body,
        in_specs=[
            pl.BlockSpec((_NCK, HID), lambda: (0, 0)),
            pl.BlockSpec((HID, HID), lambda: (0, 0)),
            pl.BlockSpec((1, HID), lambda: (0, 0)),
            pl.BlockSpec((HOPS, HID), lambda: (0, 0)),
            pl.BlockSpec((1, HID), lambda: (0, 0)),
            pl.BlockSpec((1, HOPS), lambda: (0, 0)),
        ],
        out_specs=[
            pl.BlockSpec((1, HOPS), lambda: (0, 0)),
            pl.BlockSpec((1, HOPS), lambda: (0, 0)),
        ],
        out_shape=[
            jax.ShapeDtypeStruct((1, HOPS), jnp.float32),
            jax.ShapeDtypeStruct((1, HOPS), jnp.float32),
        ],
    )(psums, W2, b2, hop_embed, wn_row, noise_row)


def kernel(subs, rels, entity_embed, relation_embed, hop_embed, W1, b1, W2,
           b2, w_n, noise_eps):
    W1b = W1.astype(jnp.bfloat16)
    b1r = b1.reshape(1, HID)
    reli = _pack_rel(relation_embed)
    psums = []
    for k in range(_NCK):
        subs_c = lax.slice(subs, (k * _BC,), ((k + 1) * _BC,))
        rels_c = lax.slice(rels, (k * _BC,), ((k + 1) * _BC,))
        sub_c, rel_c = _gather_rows(entity_embed, reli, subs_c, rels_c)
        psums.append(_tc_partial(sub_c, rel_c, W1b, b1r))
    g, q = _tc_final(jnp.concatenate(psums, axis=0), W2,
                     b2.reshape(1, HID), hop_embed, w_n.reshape(1, HID),
                     noise_eps.reshape(1, HOPS))
    return (g.reshape(HOPS), q.reshape(HOPS))
